# baseline (speedup 1.0000x reference)
import functools

import jax
import jax.numpy as jnp
from jax import lax
from jax.experimental import pallas as pl
from jax.experimental.pallas import tpu as pltpu
from jax.experimental.pallas import tpu_sc as plsc

_R0, _C0, _R1, _C1 = 1024, 1024, 3071, 3071
_N = 4096
_NC, _NS = 2, 16
_NW = _NC * _NS
_RPW = _N // _NW
_W = _C1 - _C0 + 1
_FR = 32


def _sc_kernel(b_hbm, bt_hbm, out_hbm, btv, rowv, spf, sem):
    wid = lax.axis_index("s") * _NC + lax.axis_index("c")
    sid = lax.axis_index("s")
    r0 = wid * _RPW
    interior = jnp.logical_and(r0 >= _R0, r0 <= _R1)

    @pl.when(jnp.logical_not(interior))
    def _exterior():
        pltpu.async_copy(b_hbm.at[pl.ds(r0, _RPW)],
                         out_hbm.at[pl.ds(r0, _RPW)], sem).wait()

    @pl.when(interior)
    def _interior():
        pltpu.sync_copy(bt_hbm, btv)
        v = btv[...]
        for i in range(_W // 16):
            rowv[pl.ds(i * 16, 16)] = v
        reps = [pltpu.async_copy(rowv, spf.at[sid, r], sem)
                for r in range(_FR)]
        for c in reps:
            c.wait()
        copies = [
            pltpu.async_copy(b_hbm.at[pl.ds(r0, _RPW), pl.ds(0, _C0)],
                             out_hbm.at[pl.ds(r0, _RPW), pl.ds(0, _C0)], sem),
            pltpu.async_copy(
                b_hbm.at[pl.ds(r0, _RPW), pl.ds(_C1 + 1, _N - _C1 - 1)],
                out_hbm.at[pl.ds(r0, _RPW), pl.ds(_C1 + 1, _N - _C1 - 1)],
                sem),
        ]
        for k in range(_RPW // _FR):
            copies.append(pltpu.async_copy(
                spf.at[sid],
                out_hbm.at[pl.ds(r0 + k * _FR, _FR), pl.ds(_C0, _W)], sem))
        for c in copies:
            c.wait()


@functools.partial(jax.jit, static_argnames=())
def _run(b2, bt16):
    mesh = plsc.VectorSubcoreMesh(core_axis_name="c", subcore_axis_name="s",
                                  num_cores=_NC, num_subcores=_NS)
    return pl.kernel(
        _sc_kernel,
        out_type=jax.ShapeDtypeStruct((_N, _N), jnp.float32),
        mesh=mesh,
        scratch_types=[
            pltpu.VMEM((16,), jnp.float32),
            pltpu.VMEM((_W,), jnp.float32),
            pltpu.VMEM_SHARED((_NS, _FR, _W), jnp.float32),
            pltpu.SemaphoreType.DMA,
        ],
    )(b2, bt16)


def kernel(B, Bt):
    b2 = B.reshape(_N, _N)
    bt16 = jnp.broadcast_to(Bt.reshape(1), (16,))
    return _run(b2, bt16).reshape(1, _N, _N)


# speedup vs baseline: 22.9698x; 22.9698x over previous
import functools

import jax
import jax.numpy as jnp
from jax import lax
from jax.experimental import pallas as pl
from jax.experimental.pallas import tpu as pltpu
from jax.experimental.pallas import tpu_sc as plsc

_R0, _C0, _R1, _C1 = 1024, 1024, 3071, 3071
_N = 4096
_NC, _NS = 2, 16
_NW = _NC * _NS
_RPW = _N // _NW
_W = _C1 - _C0 + 1
_FR = 32
_ECR = 8
_ICR = 16


def _sc_kernel(b_hbm, bt_hbm, out_hbm, btv, rowv, vb0, vb1, sb0, sb1, spf,
               si0, si1, so0, so1, sfill, srep):
    cid = lax.axis_index("c")
    sid = lax.axis_index("s")
    wid = sid * _NC + cid
    r0 = wid * _RPW
    interior = jnp.logical_and(r0 >= _R0, r0 <= _R1)

    @pl.when(sid == 0)
    def _build_fill():
        pltpu.sync_copy(bt_hbm, btv)
        v = btv[...]
        for i in range(_W // 16):
            rowv[pl.ds(i * 16, 16)] = v
        reps = [pltpu.async_copy(rowv, spf.at[r], srep) for r in range(_FR)]
        for c in reps:
            c.wait()

    plsc.subcore_barrier()

    @pl.when(jnp.logical_not(interior))
    def _exterior():
        n = _RPW // _ECR
        vb = [vb0, vb1]
        sin = [si0, si1]
        sout = [so0, so1]
        hin = [None, None]
        hout = [None, None]
        for k in range(2):
            hin[k] = pltpu.async_copy(
                b_hbm.at[pl.ds(r0 + k * _ECR, _ECR)], vb[k], sin[k])
        for k in range(n):
            b = k & 1
            hin[b].wait()
            hout[b] = pltpu.async_copy(
                vb[b], out_hbm.at[pl.ds(r0 + k * _ECR, _ECR)], sout[b])
            if k + 2 < n:
                hout[b].wait()
                hin[b] = pltpu.async_copy(
                    b_hbm.at[pl.ds(r0 + (k + 2) * _ECR, _ECR)], vb[b], sin[b])
        hout[0].wait()
        hout[1].wait()

    @pl.when(interior)
    def _interior():
        fills = [
            pltpu.async_copy(
                spf, out_hbm.at[pl.ds(r0 + k * _FR, _FR), pl.ds(_C0, _W)],
                sfill)
            for k in range(_RPW // _FR)
        ]
        cols = [0, _C1 + 1]
        chunks = [(r0 + (j // 2) * _ICR, cols[j & 1])
                  for j in range(2 * (_RPW // _ICR))]
        n = len(chunks)
        sb = [sb0, sb1]
        sin = [si0, si1]
        sout = [so0, so1]
        hin = [None, None]
        hout = [None, None]
        for k in range(2):
            rr, cc = chunks[k]
            hin[k] = pltpu.async_copy(
                b_hbm.at[pl.ds(rr, _ICR), pl.ds(cc, _C0)], sb[k], sin[k])
        for k in range(n):
            b = k & 1
            rr, cc = chunks[k]
            hin[b].wait()
            hout[b] = pltpu.async_copy(
                sb[b], out_hbm.at[pl.ds(rr, _ICR), pl.ds(cc, _C0)], sout[b])
            if k + 2 < n:
                hout[b].wait()
                rr2, cc2 = chunks[k + 2]
                hin[b] = pltpu.async_copy(
                    b_hbm.at[pl.ds(rr2, _ICR), pl.ds(cc2, _C0)], sb[b], sin[b])
        hout[0].wait()
        hout[1].wait()
        for c in fills:
            c.wait()


@jax.jit
def _run(b2, bt16):
    mesh = plsc.VectorSubcoreMesh(core_axis_name="c", subcore_axis_name="s",
                                  num_cores=_NC, num_subcores=_NS)
    return pl.kernel(
        _sc_kernel,
        out_type=jax.ShapeDtypeStruct((_N, _N), jnp.float32),
        mesh=mesh,
        scratch_types=[
            pltpu.VMEM((16,), jnp.float32),
            pltpu.VMEM((_W,), jnp.float32),
            pltpu.VMEM((_ECR, _N), jnp.float32),
            pltpu.VMEM((_ECR, _N), jnp.float32),
            pltpu.VMEM((_ICR, _C0), jnp.float32),
            pltpu.VMEM((_ICR, _C0), jnp.float32),
            pltpu.VMEM_SHARED((_FR, _W), jnp.float32),
            pltpu.SemaphoreType.DMA,
            pltpu.SemaphoreType.DMA,
            pltpu.SemaphoreType.DMA,
            pltpu.SemaphoreType.DMA,
            pltpu.SemaphoreType.DMA,
            pltpu.SemaphoreType.DMA,
        ],
    )(b2, bt16)


def kernel(B, Bt):
    b2 = B.reshape(_N, _N)
    bt16 = jnp.broadcast_to(Bt.reshape(1), (16,))
    return _run(b2, bt16).reshape(1, _N, _N)


# speedup vs baseline: 23.3174x; 1.0151x over previous
import jax
import jax.numpy as jnp
from jax import lax
from jax.experimental import pallas as pl
from jax.experimental.pallas import tpu as pltpu
from jax.experimental.pallas import tpu_sc as plsc

_R0, _C0, _R1, _C1 = 1024, 1024, 3071, 3071
_N = 4096
_NC, _NS = 2, 16
_NW = _NC * _NS
_RPW = _N // _NW
_W = _C1 - _C0 + 1
_FR = 8
_ECR = 4
_ICR = 16


def _sc_kernel(b_hbm, bt_hbm, out_hbm, btv, fillv, vb0, vb1, sb0, sb1,
               si0, si1, so0, so1, sfill):
    cid = lax.axis_index("c")
    sid = lax.axis_index("s")
    wid = sid * _NC + cid
    r0 = wid * _RPW
    interior = jnp.logical_and(r0 >= _R0, r0 <= _R1)

    @pl.when(jnp.logical_not(interior))
    def _exterior():
        n = _RPW // _ECR
        vb = [vb0, vb1]
        sin = [si0, si1]
        sout = [so0, so1]
        hin = [None, None]
        hout = [None, None]
        for k in range(2):
            hin[k] = pltpu.async_copy(
                b_hbm.at[pl.ds(r0 + k * _ECR, _ECR)], vb[k], sin[k])
        for k in range(n):
            b = k & 1
            hin[b].wait()
            hout[b] = pltpu.async_copy(
                vb[b], out_hbm.at[pl.ds(r0 + k * _ECR, _ECR)], sout[b])
            if k + 2 < n:
                hout[b].wait()
                hin[b] = pltpu.async_copy(
                    b_hbm.at[pl.ds(r0 + (k + 2) * _ECR, _ECR)], vb[b], sin[b])
        hout[0].wait()
        hout[1].wait()

    @pl.when(interior)
    def _interior():
        pltpu.sync_copy(bt_hbm, btv)
        v = btv[...]
        for r in range(_FR):
            for i in range(_W // 16):
                fillv[r, pl.ds(i * 16, 16)] = v
        fills = [
            pltpu.async_copy(
                fillv, out_hbm.at[pl.ds(r0 + k * _FR, _FR), pl.ds(_C0, _W)],
                sfill)
            for k in range(_RPW // _FR)
        ]
        cols = [0, _C1 + 1]
        chunks = [(r0 + (j // 2) * _ICR, cols[j & 1])
                  for j in range(2 * (_RPW // _ICR))]
        n = len(chunks)
        sb = [sb0, sb1]
        sin = [si0, si1]
        sout = [so0, so1]
        hin = [None, None]
        hout = [None, None]
        for k in range(2):
            rr, cc = chunks[k]
            hin[k] = pltpu.async_copy(
                b_hbm.at[pl.ds(rr, _ICR), pl.ds(cc, _C0)], sb[k], sin[k])
        for k in range(n):
            b = k & 1
            rr, cc = chunks[k]
            hin[b].wait()
            hout[b] = pltpu.async_copy(
                sb[b], out_hbm.at[pl.ds(rr, _ICR), pl.ds(cc, _C0)], sout[b])
            if k + 2 < n:
                hout[b].wait()
                rr2, cc2 = chunks[k + 2]
                hin[b] = pltpu.async_copy(
                    b_hbm.at[pl.ds(rr2, _ICR), pl.ds(cc2, _C0)], sb[b], sin[b])
        hout[0].wait()
        hout[1].wait()
        for c in fills:
            c.wait()


@jax.jit
def _run(b2, bt16):
    mesh = plsc.VectorSubcoreMesh(core_axis_name="c", subcore_axis_name="s",
                                  num_cores=_NC, num_subcores=_NS)
    return pl.kernel(
        _sc_kernel,
        out_type=jax.ShapeDtypeStruct((_N, _N), jnp.float32),
        mesh=mesh,
        scratch_types=[
            pltpu.VMEM((16,), jnp.float32),
            pltpu.VMEM((_FR, _W), jnp.float32),
            pltpu.VMEM((_ECR, _N), jnp.float32),
            pltpu.VMEM((_ECR, _N), jnp.float32),
            pltpu.VMEM((_ICR, _C0), jnp.float32),
            pltpu.VMEM((_ICR, _C0), jnp.float32),
            pltpu.SemaphoreType.DMA,
            pltpu.SemaphoreType.DMA,
            pltpu.SemaphoreType.DMA,
            pltpu.SemaphoreType.DMA,
            pltpu.SemaphoreType.DMA,
        ],
    )(b2, bt16)


def kernel(B, Bt):
    b2 = B.reshape(_N, _N)
    bt16 = jnp.broadcast_to(Bt.reshape(1), (16,))
    return _run(b2, bt16).reshape(1, _N, _N)


# speedup vs baseline: 23.8664x; 1.0235x over previous
import jax
import jax.numpy as jnp
from jax import lax
from jax.experimental import pallas as pl
from jax.experimental.pallas import tpu as pltpu
from jax.experimental.pallas import tpu_sc as plsc

_R0, _C0, _R1, _C1 = 1024, 1024, 3071, 3071
_N = 4096
_NC, _NS = 2, 16
_NW = _NC * _NS
_RPW = _N // _NW
_W = _C1 - _C0 + 1
_FR = 8
_ECR = 4
_ICR = 16


def _sc_kernel(b_hbm, bt_hbm, out_hbm, btv, fillv, vb0, vb1, vb2, sb0, sb1,
               sb2, si0, si1, si2, so0, so1, so2, sfill):
    cid = lax.axis_index("c")
    sid = lax.axis_index("s")
    wid = sid * _NC + cid
    r0 = wid * _RPW
    interior = jnp.logical_and(r0 >= _R0, r0 <= _R1)

    @pl.when(jnp.logical_not(interior))
    def _exterior():
        n = _RPW // _ECR
        vb = [vb0, vb1, vb2]
        sin = [si0, si1, si2]
        sout = [so0, so1, so2]
        hin = [None] * 3
        hout = [None] * 3
        for k in range(3):
            hin[k] = pltpu.async_copy(
                b_hbm.at[pl.ds(r0 + k * _ECR, _ECR)], vb[k], sin[k])
        for k in range(n):
            b = k % 3
            hin[b].wait()
            hout[b] = pltpu.async_copy(
                vb[b], out_hbm.at[pl.ds(r0 + k * _ECR, _ECR)], sout[b])
            if k + 3 < n:
                hout[b].wait()
                hin[b] = pltpu.async_copy(
                    b_hbm.at[pl.ds(r0 + (k + 3) * _ECR, _ECR)], vb[b], sin[b])
        hout[(n - 3) % 3].wait()
        hout[(n - 2) % 3].wait()
        hout[(n - 1) % 3].wait()

    @pl.when(interior)
    def _interior():
        pltpu.sync_copy(bt_hbm, btv)
        v = btv[...]
        for r in range(_FR):
            for i in range(_W // 16):
                fillv[r, pl.ds(i * 16, 16)] = v
        fills = [
            pltpu.async_copy(
                fillv, out_hbm.at[pl.ds(r0 + k * _FR, _FR), pl.ds(_C0, _W)],
                sfill)
            for k in range(_RPW // _FR)
        ]
        cols = [0, _C1 + 1]
        chunks = [(r0 + (j // 2) * _ICR, cols[j & 1])
                  for j in range(2 * (_RPW // _ICR))]
        n = len(chunks)
        sb = [sb0, sb1, sb2]
        sin = [si0, si1, si2]
        sout = [so0, so1, so2]
        hin = [None] * 3
        hout = [None] * 3
        for k in range(3):
            rr, cc = chunks[k]
            hin[k] = pltpu.async_copy(
                b_hbm.at[pl.ds(rr, _ICR), pl.ds(cc, _C0)], sb[k], sin[k])
        for k in range(n):
            b = k % 3
            rr, cc = chunks[k]
            hin[b].wait()
            hout[b] = pltpu.async_copy(
                sb[b], out_hbm.at[pl.ds(rr, _ICR), pl.ds(cc, _C0)], sout[b])
            if k + 3 < n:
                hout[b].wait()
                rr2, cc2 = chunks[k + 3]
                hin[b] = pltpu.async_copy(
                    b_hbm.at[pl.ds(rr2, _ICR), pl.ds(cc2, _C0)], sb[b], sin[b])
        hout[(n - 3) % 3].wait()
        hout[(n - 2) % 3].wait()
        hout[(n - 1) % 3].wait()
        for c in fills:
            c.wait()


@jax.jit
def _run(b2, bt16):
    mesh = plsc.VectorSubcoreMesh(core_axis_name="c", subcore_axis_name="s",
                                  num_cores=_NC, num_subcores=_NS)
    return pl.kernel(
        _sc_kernel,
        out_type=jax.ShapeDtypeStruct((_N, _N), jnp.float32),
        mesh=mesh,
        scratch_types=[
            pltpu.VMEM((16,), jnp.float32),
            pltpu.VMEM((_FR, _W), jnp.float32),
            pltpu.VMEM((_ECR, _N), jnp.float32),
            pltpu.VMEM((_ECR, _N), jnp.float32),
            pltpu.VMEM((_ECR, _N), jnp.float32),
            pltpu.VMEM((_ICR, _C0), jnp.float32),
            pltpu.VMEM((_ICR, _C0), jnp.float32),
            pltpu.VMEM((_ICR, _C0), jnp.float32),
            pltpu.SemaphoreType.DMA,
            pltpu.SemaphoreType.DMA,
            pltpu.SemaphoreType.DMA,
            pltpu.SemaphoreType.DMA,
            pltpu.SemaphoreType.DMA,
            pltpu.SemaphoreType.DMA,
            pltpu.SemaphoreType.DMA,
        ],
    )(b2, bt16)


def kernel(B, Bt):
    b2 = B.reshape(_N, _N)
    bt16 = jnp.broadcast_to(Bt.reshape(1), (16,))
    return _run(b2, bt16).reshape(1, _N, _N)


# speedup vs baseline: 23.8966x; 1.0013x over previous
import jax
import jax.numpy as jnp
from jax import lax
from jax.experimental import pallas as pl
from jax.experimental.pallas import tpu as pltpu
from jax.experimental.pallas import tpu_sc as plsc

_R0, _C0, _R1, _C1 = 1024, 1024, 3071, 3071
_N = 4096
_NC, _NS = 2, 16
_NW = _NC * _NS
_RPW = _N // _NW
_W = _C1 - _C0 + 1
_FR = 8
_ECR = 4
_ICR = 16


def _sc_kernel(b_hbm, bt_hbm, out_hbm, btv, fillv, vb0, vb1, vb2, sb0, sb1,
               sb2, si0, si1, si2, so0, so1, so2, sfill):
    cid = lax.axis_index("c")
    sid = lax.axis_index("s")
    wid = sid * _NC + cid
    r0 = wid * _RPW
    interior = jnp.logical_and(r0 >= _R0, r0 <= _R1)

    @pl.when(jnp.logical_not(interior))
    def _exterior():
        n = _RPW // _ECR
        vb = [vb0, vb1, vb2]
        sin = [si0, si1, si2]
        sout = [so0, so1, so2]
        hin = [None] * 3
        hout = [None] * 3
        for k in range(3):
            hin[k] = pltpu.async_copy(
                b_hbm.at[pl.ds(r0 + k * _ECR, _ECR)], vb[k], sin[k])
        for k in range(n):
            b = k % 3
            hin[b].wait()
            hout[b] = pltpu.async_copy(
                vb[b], out_hbm.at[pl.ds(r0 + k * _ECR, _ECR)], sout[b])
            if k + 3 < n:
                hout[b].wait()
                hin[b] = pltpu.async_copy(
                    b_hbm.at[pl.ds(r0 + (k + 3) * _ECR, _ECR)], vb[b], sin[b])
        hout[(n - 3) % 3].wait()
        hout[(n - 2) % 3].wait()
        hout[(n - 1) % 3].wait()

    @pl.when(interior)
    def _interior():
        cols = [0, _C1 + 1]
        chunks = [(r0 + (j // 2) * _ICR, cols[j & 1])
                  for j in range(2 * (_RPW // _ICR))]
        n = len(chunks)
        sb = [sb0, sb1, sb2]
        sin = [si0, si1, si2]
        sout = [so0, so1, so2]
        hin = [None] * 3
        hout = [None] * 3
        for k in range(3):
            rr, cc = chunks[k]
            hin[k] = pltpu.async_copy(
                b_hbm.at[pl.ds(rr, _ICR), pl.ds(cc, _C0)], sb[k], sin[k])
        pltpu.sync_copy(bt_hbm, btv)
        v = btv[...]
        for r in range(_FR):
            for i in range(_W // 16):
                fillv[r, pl.ds(i * 16, 16)] = v
        fills = [
            pltpu.async_copy(
                fillv, out_hbm.at[pl.ds(r0 + k * _FR, _FR), pl.ds(_C0, _W)],
                sfill)
            for k in range(_RPW // _FR)
        ]
        for k in range(n):
            b = k % 3
            rr, cc = chunks[k]
            hin[b].wait()
            hout[b] = pltpu.async_copy(
                sb[b], out_hbm.at[pl.ds(rr, _ICR), pl.ds(cc, _C0)], sout[b])
            if k + 3 < n:
                hout[b].wait()
                rr2, cc2 = chunks[k + 3]
                hin[b] = pltpu.async_copy(
                    b_hbm.at[pl.ds(rr2, _ICR), pl.ds(cc2, _C0)], sb[b], sin[b])
        hout[(n - 3) % 3].wait()
        hout[(n - 2) % 3].wait()
        hout[(n - 1) % 3].wait()
        for c in fills:
            c.wait()


@jax.jit
def _run(b2, bt16):
    mesh = plsc.VectorSubcoreMesh(core_axis_name="c", subcore_axis_name="s",
                                  num_cores=_NC, num_subcores=_NS)
    return pl.kernel(
        _sc_kernel,
        out_type=jax.ShapeDtypeStruct((_N, _N), jnp.float32),
        mesh=mesh,
        scratch_types=[
            pltpu.VMEM((16,), jnp.float32),
            pltpu.VMEM((_FR, _W), jnp.float32),
            pltpu.VMEM((_ECR, _N), jnp.float32),
            pltpu.VMEM((_ECR, _N), jnp.float32),
            pltpu.VMEM((_ECR, _N), jnp.float32),
            pltpu.VMEM((_ICR, _C0), jnp.float32),
            pltpu.VMEM((_ICR, _C0), jnp.float32),
            pltpu.VMEM((_ICR, _C0), jnp.float32),
            pltpu.SemaphoreType.DMA,
            pltpu.SemaphoreType.DMA,
            pltpu.SemaphoreType.DMA,
            pltpu.SemaphoreType.DMA,
            pltpu.SemaphoreType.DMA,
            pltpu.SemaphoreType.DMA,
            pltpu.SemaphoreType.DMA,
        ],
    )(b2, bt16)


def kernel(B, Bt):
    b2 = B.reshape(_N, _N)
    bt16 = jnp.broadcast_to(Bt.reshape(1), (16,))
    return _run(b2, bt16).reshape(1, _N, _N)
